# Initial kernel scaffold; baseline (speedup 1.0000x reference)
import jax
import jax.numpy as jnp
from jax.experimental import pallas as pl


def kernel(x, edge_index, edge_attr, Wq, Wk, Wv, We):
    raise NotImplementedError("write your pallas kernel here")


# speedup vs baseline: 2.7557x; 2.7557x over previous
import functools
import math

import jax
import jax.numpy as jnp
from jax import lax
from jax.experimental import pallas as pl
from jax.experimental.pallas import tpu as pltpu
from jax.experimental.pallas import tpu_sc as plsc

N = 10000
E = 320000
D = 128
QW = 144
KVW = 256
AW = 144
NC, NS, L = 2, 16, 16
AGG_N = 10240
NW = NC * NS
B = 64
NCHUNK = E // B


def _prep_body(h_ref, wq_ref, wk_ref, wv_ref, wet_ref, qp_ref, kv_ref):
    h = h_ref[...]
    q = jnp.dot(h, wq_ref[...], preferred_element_type=jnp.float32)
    q = q * (1.0 / math.sqrt(D))
    qp_ref[:, :D] = q
    qp_ref[:, D:] = jnp.dot(q, wet_ref[...], preferred_element_type=jnp.float32)
    kv_ref[:, :D] = jnp.dot(h, wk_ref[...], preferred_element_type=jnp.float32)
    kv_ref[:, D:] = jnp.dot(h, wv_ref[...], preferred_element_type=jnp.float32)


def _prep(h, wq, wk, wv, wet):
    R = 1000
    return pl.pallas_call(
        _prep_body,
        grid=(N // R,),
        in_specs=[
            pl.BlockSpec((R, D), lambda i: (i, 0)),
            pl.BlockSpec((D, D), lambda i: (0, 0)),
            pl.BlockSpec((D, D), lambda i: (0, 0)),
            pl.BlockSpec((D, D), lambda i: (0, 0)),
            pl.BlockSpec((D, QW - D), lambda i: (0, 0)),
        ],
        out_specs=[
            pl.BlockSpec((R, QW), lambda i: (i, 0)),
            pl.BlockSpec((R, KVW), lambda i: (i, 0)),
        ],
        out_shape=[
            jax.ShapeDtypeStruct((N, QW), jnp.float32),
            jax.ShapeDtypeStruct((N, KVW), jnp.float32),
        ],
    )(h, wq, wk, wv, wet)


def _combine_body(acc_ref, out_ref, *, relu):
    a = acc_ref[0] + acc_ref[1]
    den = a[:, D:D + 1] + 1e-30
    h = a[:, :D] / den
    if relu:
        h = jnp.maximum(h, 0.0)
    out_ref[...] = h


def _combine(acc, relu):
    R = 1000
    return pl.pallas_call(
        functools.partial(_combine_body, relu=relu),
        grid=(N // R,),
        in_specs=[pl.BlockSpec((NC, R, AW), lambda i: (0, i, 0))],
        out_specs=pl.BlockSpec((R, D), lambda i: (i, 0)),
        out_shape=jax.ShapeDtypeStruct((N, D), jnp.float32),
    )(acc)


def _sc_body(qp_hbm, kv_hbm, src_hbm, dst_hbm, ea_hbm, out_hbm,
             agg_sh, srcb, dstb, eab, qrows, kvrows, w, sem):
    cid = lax.axis_index("c")
    sid = lax.axis_index("s")
    wid = sid * NC + cid

    @pl.loop(0, B)
    def _zero_w(i):
        for j in range(AW // L):
            w[i, pl.ds(j * L, L)] = jnp.zeros((L,), jnp.float32)

    row0 = sid * (AGG_N // NS)
    for c in range(AGG_N // NS // B):
        pltpu.sync_copy(w, agg_sh.at[pl.ds(row0 + c * B, B)])

    plsc.subcore_barrier()

    lanes = lax.broadcasted_iota(jnp.int32, (L,), 0)
    nchunk = jnp.where(wid < NCHUNK - (NCHUNK // NW) * NW, NCHUNK // NW + 1,
                       NCHUNK // NW)

    @pl.loop(0, nchunk)
    def _chunk(t):
        base = (wid + t * NW) * B
        pltpu.sync_copy(src_hbm.at[pl.ds(base, B)], srcb)
        pltpu.sync_copy(dst_hbm.at[pl.ds(base, B)], dstb)
        pltpu.sync_copy(ea_hbm.at[pl.ds(base, B)], eab)
        cp_q = pltpu.async_copy(qp_hbm.at[dstb], qrows, sem)
        cp_kv = pltpu.async_copy(kv_hbm.at[srcb], kvrows, sem)
        cp_q.wait()
        cp_kv.wait()

        for g in range(B // L):
            rows = lanes + g * L

            @pl.loop(0, D, init_carry=jnp.zeros((L,), jnp.float32), unroll=8)
            def _dot(d, acc):
                col = jnp.full((L,), d, jnp.int32)
                qd = plsc.load_gather(qrows, [rows, col])
                kd = plsc.load_gather(kvrows, [rows, col])
                return acc + qd * kd

            acc = _dot
            qe0 = plsc.load_gather(qrows, [rows, jnp.full((L,), D, jnp.int32)])
            qe1 = plsc.load_gather(qrows, [rows, jnp.full((L,), D + 1, jnp.int32)])
            ea0 = plsc.load_gather(eab, [rows, jnp.zeros((L,), jnp.int32)])
            ea1 = plsc.load_gather(eab, [rows, jnp.full((L,), 1, jnp.int32)])
            s = jnp.exp(acc + qe0 * ea0 + qe1 * ea1)
            plsc.store_scatter(w, [rows, jnp.full((L,), D, jnp.int32)], s)

            @pl.loop(0, D, unroll=8)
            def _wcols(d):
                vd = plsc.load_gather(kvrows, [rows, jnp.full((L,), D + d, jnp.int32)])
                plsc.store_scatter(w, [rows, jnp.full((L,), d, jnp.int32)], s * vd)

        pltpu.sync_copy(w, agg_sh.at[dstb], add=True)

    plsc.subcore_barrier()
    pltpu.sync_copy(agg_sh.at[pl.ds(row0, AGG_N // NS)],
                    out_hbm.at[cid, pl.ds(row0, AGG_N // NS)])


@functools.partial(
    pl.kernel,
    out_type=jax.ShapeDtypeStruct((NC, AGG_N, AW), jnp.float32),
    mesh=plsc.VectorSubcoreMesh(core_axis_name="c", subcore_axis_name="s",
                                num_cores=NC, num_subcores=NS),
    compiler_params=pltpu.CompilerParams(use_tc_tiling_on_sc=False,
                                         needs_layout_passes=False),
    scratch_types=[
        pltpu.VMEM_SHARED((AGG_N, AW), jnp.float32),
        pltpu.VMEM((B,), jnp.int32),
        pltpu.VMEM((B,), jnp.int32),
        pltpu.VMEM((B, 2), jnp.float32),
        pltpu.VMEM((B, QW), jnp.float32),
        pltpu.VMEM((B, KVW), jnp.float32),
        pltpu.VMEM((B, AW), jnp.float32),
        pltpu.SemaphoreType.DMA,
    ],
)
def _sc_layer(qp_hbm, kv_hbm, src_hbm, dst_hbm, ea_hbm, out_hbm,
              agg_sh, srcb, dstb, eab, qrows, kvrows, w, sem):
    _sc_body(qp_hbm, kv_hbm, src_hbm, dst_hbm, ea_hbm, out_hbm,
             agg_sh, srcb, dstb, eab, qrows, kvrows, w, sem)


def kernel(x, edge_index, edge_attr, Wq, Wk, Wv, We):
    src = edge_index[0]
    dst = edge_index[1]
    n_layers = Wq.shape[0]
    h = x
    for l in range(n_layers):
        wet = jnp.zeros((D, QW - D), jnp.float32).at[:, :2].set(We[l].T)
        qp, kv = _prep(h, Wq[l], Wk[l], Wv[l], wet)
        acc = _sc_layer(qp, kv, src, dst, edge_attr)
        h = _combine(acc, relu=(l < n_layers - 1))
    return h


# speedup vs baseline: 3.3500x; 1.2157x over previous
import functools
import math

import jax
import jax.numpy as jnp
from jax import lax
from jax.experimental import pallas as pl
from jax.experimental.pallas import tpu as pltpu
from jax.experimental.pallas import tpu_sc as plsc

N = 10000
E = 320000
D = 128
QW = 144
KVW = 256
AW = 144
NC, NS, L = 2, 16, 16
AGG_N = 10240
NW = NC * NS
B = 32
NCHUNK = E // B
FULL_T = NCHUNK // NW
EXTRA0 = FULL_T * NW


def _prep_body(h_ref, wq_ref, wk_ref, wv_ref, wet_ref, qp_ref, kv_ref):
    h = h_ref[...]
    q = jnp.dot(h, wq_ref[...], preferred_element_type=jnp.float32)
    q = q * (1.0 / math.sqrt(D))
    qp_ref[:, :D] = q
    qp_ref[:, D:] = jnp.dot(q, wet_ref[...], preferred_element_type=jnp.float32)
    kv_ref[:, :D] = jnp.dot(h, wk_ref[...], preferred_element_type=jnp.float32)
    kv_ref[:, D:] = jnp.dot(h, wv_ref[...], preferred_element_type=jnp.float32)


def _prep(h, wq, wk, wv, wet):
    R = 1000
    return pl.pallas_call(
        _prep_body,
        grid=(N // R,),
        in_specs=[
            pl.BlockSpec((R, D), lambda i: (i, 0)),
            pl.BlockSpec((D, D), lambda i: (0, 0)),
            pl.BlockSpec((D, D), lambda i: (0, 0)),
            pl.BlockSpec((D, D), lambda i: (0, 0)),
            pl.BlockSpec((D, QW - D), lambda i: (0, 0)),
        ],
        out_specs=[
            pl.BlockSpec((R, QW), lambda i: (i, 0)),
            pl.BlockSpec((R, KVW), lambda i: (i, 0)),
        ],
        out_shape=[
            jax.ShapeDtypeStruct((N, QW), jnp.float32),
            jax.ShapeDtypeStruct((N, KVW), jnp.float32),
        ],
    )(h, wq, wk, wv, wet)


def _combine_body(acc_ref, out_ref, *, relu):
    a = acc_ref[0] + acc_ref[1]
    den = a[:, D:D + 1] + 1e-30
    h = a[:, :D] / den
    if relu:
        h = jnp.maximum(h, 0.0)
    out_ref[...] = h


def _combine(acc, relu):
    R = 1000
    return pl.pallas_call(
        functools.partial(_combine_body, relu=relu),
        grid=(N // R,),
        in_specs=[pl.BlockSpec((NC, R, AW), lambda i: (0, i, 0))],
        out_specs=pl.BlockSpec((R, D), lambda i: (i, 0)),
        out_shape=jax.ShapeDtypeStruct((N, D), jnp.float32),
    )(acc)


def _idx_fires(src_hbm, dst_hbm, ea_hbm, srcb3, dstb3, eab3, sem_idx, slot, base):
    pltpu.async_copy(src_hbm.at[pl.ds(base, B)], srcb3.at[slot], sem_idx)
    pltpu.async_copy(dst_hbm.at[pl.ds(base, B)], dstb3.at[slot], sem_idx)
    pltpu.async_copy(ea_hbm.at[pl.ds(base, B)], eab3.at[slot], sem_idx)


def _idx_waits(src_hbm, dst_hbm, ea_hbm, srcb3, dstb3, eab3, sem_idx, slot):
    pltpu.make_async_copy(src_hbm.at[pl.ds(0, B)], srcb3.at[slot], sem_idx).wait()
    pltpu.make_async_copy(dst_hbm.at[pl.ds(0, B)], dstb3.at[slot], sem_idx).wait()
    pltpu.make_async_copy(ea_hbm.at[pl.ds(0, B)], eab3.at[slot], sem_idx).wait()


def _sc_body(qp_hbm, kv_hbm, src_hbm, dst_hbm, ea_hbm, out_hbm,
             agg_sh, srcb3, dstb3, eab3, qr2, kr2, w, sem_idx, sem_row):
    cid = lax.axis_index("c")
    sid = lax.axis_index("s")
    wid = sid * NC + cid

    @pl.loop(0, B)
    def _zero_w(i):
        for j in range(AW // L):
            w[i, pl.ds(j * L, L)] = jnp.zeros((L,), jnp.float32)

    row0 = sid * (AGG_N // NS)
    for c in range(AGG_N // NS // B):
        pltpu.sync_copy(w, agg_sh.at[pl.ds(row0 + c * B, B)])

    plsc.subcore_barrier()

    lanes = lax.broadcasted_iota(jnp.int32, (L,), 0)
    nchunk = jnp.where(wid < NCHUNK - EXTRA0, FULL_T + 1, FULL_T)

    def chunk_id(t):
        return jnp.where(t < FULL_T, wid + t * NW, EXTRA0 + wid)

    def compute_chunk(qr, kr, eab):
        for g in range(B // L):
            rows = lanes + g * L

            @pl.loop(0, D, init_carry=(jnp.zeros((L,), jnp.float32), lanes * 0),
                     unroll=8)
            def _dot(d, carry):
                acc, colv = carry
                qd = plsc.load_gather(qr, [rows, colv])
                kd = plsc.load_gather(kr, [rows, colv])
                return acc + qd * kd, colv + 1

            acc, _ = _dot
            qe0 = plsc.load_gather(qr, [rows, jnp.full((L,), D, jnp.int32)])
            qe1 = plsc.load_gather(qr, [rows, jnp.full((L,), D + 1, jnp.int32)])
            ea0 = plsc.load_gather(eab, [rows, jnp.zeros((L,), jnp.int32)])
            ea1 = plsc.load_gather(eab, [rows, jnp.full((L,), 1, jnp.int32)])
            s = jnp.exp(acc + qe0 * ea0 + qe1 * ea1)
            plsc.store_scatter(w, [rows, jnp.full((L,), D, jnp.int32)], s)

            @pl.loop(0, D, init_carry=(lanes * 0 + D, lanes * 0), unroll=8)
            def _wcols(d, carry):
                vcol, wcol = carry
                vd = plsc.load_gather(kr, [rows, vcol])
                plsc.store_scatter(w, [rows, wcol], s * vd)
                return vcol + 1, wcol + 1

    pltpu.sync_copy(src_hbm.at[pl.ds(wid * B, B)], srcb3.at[0])
    pltpu.sync_copy(dst_hbm.at[pl.ds(wid * B, B)], dstb3.at[0])
    pltpu.sync_copy(ea_hbm.at[pl.ds(wid * B, B)], eab3.at[0])
    pltpu.async_copy(qp_hbm.at[dstb3.at[0]], qr2[0], sem_row)
    pltpu.async_copy(kv_hbm.at[srcb3.at[0]], kr2[0], sem_row)
    _idx_fires(src_hbm, dst_hbm, ea_hbm, srcb3, dstb3, eab3, sem_idx, 1,
               chunk_id(1) * B)

    n_outer = (FULL_T + 1 + 5) // 6

    @pl.loop(0, n_outer)
    def _outer(t6):
        for i in range(6):
            t = t6 * 6 + i
            p2, p3 = i % 2, i % 3
            n2, n3 = (i + 1) % 2, (i + 1) % 3
            m3 = (i + 2) % 3

            @pl.when(t + 1 < nchunk)
            def _():
                _idx_waits(src_hbm, dst_hbm, ea_hbm, srcb3, dstb3, eab3,
                           sem_idx, n3)
                pltpu.async_copy(qp_hbm.at[dstb3.at[n3]], qr2[n2], sem_row)
                pltpu.async_copy(kv_hbm.at[srcb3.at[n3]], kr2[n2], sem_row)

            @pl.when(t + 2 < nchunk)
            def _():
                _idx_fires(src_hbm, dst_hbm, ea_hbm, srcb3, dstb3, eab3,
                           sem_idx, m3, chunk_id(t + 2) * B)

            @pl.when(t < nchunk)
            def _():
                pltpu.make_async_copy(qp_hbm.at[dstb3.at[p3]], qr2[p2],
                                      sem_row).wait()
                pltpu.make_async_copy(kv_hbm.at[srcb3.at[p3]], kr2[p2],
                                      sem_row).wait()
                compute_chunk(qr2[p2], kr2[p2], eab3.at[p3])
                pltpu.sync_copy(w, agg_sh.at[dstb3.at[p3]], add=True)

    plsc.subcore_barrier()
    pltpu.sync_copy(agg_sh.at[pl.ds(row0, AGG_N // NS)],
                    out_hbm.at[cid, pl.ds(row0, AGG_N // NS)])


@functools.partial(
    pl.kernel,
    out_type=jax.ShapeDtypeStruct((NC, AGG_N, AW), jnp.float32),
    mesh=plsc.VectorSubcoreMesh(core_axis_name="c", subcore_axis_name="s",
                                num_cores=NC, num_subcores=NS),
    compiler_params=pltpu.CompilerParams(use_tc_tiling_on_sc=False,
                                         needs_layout_passes=False,
                                         disable_bounds_checks=True),
    scratch_types=[
        pltpu.VMEM_SHARED((AGG_N, AW), jnp.float32),
        pltpu.VMEM((3, B), jnp.int32),
        pltpu.VMEM((3, B), jnp.int32),
        pltpu.VMEM((3, B, 2), jnp.float32),
        pltpu.VMEM((B, QW), jnp.float32),
        pltpu.VMEM((B, QW), jnp.float32),
        pltpu.VMEM((B, KVW), jnp.float32),
        pltpu.VMEM((B, KVW), jnp.float32),
        pltpu.VMEM((B, AW), jnp.float32),
        pltpu.SemaphoreType.DMA,
        pltpu.SemaphoreType.DMA,
    ],
)
def _sc_layer(qp_hbm, kv_hbm, src_hbm, dst_hbm, ea_hbm, out_hbm,
              agg_sh, srcb3, dstb3, eab3, qra, qrb, kra, krb, w,
              sem_idx, sem_row):
    _sc_body(qp_hbm, kv_hbm, src_hbm, dst_hbm, ea_hbm, out_hbm,
             agg_sh, srcb3, dstb3, eab3, [qra, qrb], [kra, krb], w,
             sem_idx, sem_row)


def kernel(x, edge_index, edge_attr, Wq, Wk, Wv, We):
    src = edge_index[0]
    dst = edge_index[1]
    n_layers = Wq.shape[0]
    h = x
    for l in range(n_layers):
        wet = jnp.zeros((D, QW - D), jnp.float32).at[:, :2].set(We[l].T)
        qp, kv = _prep(h, Wq[l], Wk[l], Wv[l], wet)
        acc = _sc_layer(qp, kv, src, dst, edge_attr)
        h = _combine(acc, relu=(l < n_layers - 1))
    return h


# speedup vs baseline: 12.0365x; 3.5930x over previous
import functools
import math

import jax
import jax.numpy as jnp
from jax import lax
from jax.experimental import pallas as pl
from jax.experimental.pallas import tpu as pltpu
from jax.experimental.pallas import tpu_sc as plsc

N = 10000
E = 320000
D = 128
QW = 144
KVW = 256
AW = 144
NC, NS, L = 2, 16, 16
AGG_N = 10240
NW = NC * NS
B = 32
NCHUNK = E // B
FULL_T = NCHUNK // NW
EXTRA0 = FULL_T * NW


def _prep_body(h_ref, wq_ref, wk_ref, wv_ref, wet_ref, qp_ref, kv_ref):
    h = h_ref[...]
    q = jnp.dot(h, wq_ref[...], preferred_element_type=jnp.float32)
    q = q * (1.0 / math.sqrt(D))
    qp_ref[:, :D] = q
    qp_ref[:, D:] = jnp.dot(q, wet_ref[...], preferred_element_type=jnp.float32)
    kv_ref[:, :D] = jnp.dot(h, wk_ref[...], preferred_element_type=jnp.float32)
    kv_ref[:, D:] = jnp.dot(h, wv_ref[...], preferred_element_type=jnp.float32)


def _prep(h, wq, wk, wv, wet):
    R = 1000
    return pl.pallas_call(
        _prep_body,
        grid=(N // R,),
        in_specs=[
            pl.BlockSpec((R, D), lambda i: (i, 0)),
            pl.BlockSpec((D, D), lambda i: (0, 0)),
            pl.BlockSpec((D, D), lambda i: (0, 0)),
            pl.BlockSpec((D, D), lambda i: (0, 0)),
            pl.BlockSpec((D, QW - D), lambda i: (0, 0)),
        ],
        out_specs=[
            pl.BlockSpec((R, QW), lambda i: (i, 0)),
            pl.BlockSpec((R, KVW), lambda i: (i, 0)),
        ],
        out_shape=[
            jax.ShapeDtypeStruct((N, QW), jnp.float32),
            jax.ShapeDtypeStruct((N, KVW), jnp.float32),
        ],
    )(h, wq, wk, wv, wet)


def _combine_body(acc_ref, out_ref, *, relu):
    a = acc_ref[0] + acc_ref[1]
    den = a[:, D:D + 1] + 1e-30
    h = a[:, :D] / den
    if relu:
        h = jnp.maximum(h, 0.0)
    out_ref[...] = h


def _combine(acc, relu):
    R = 1000
    return pl.pallas_call(
        functools.partial(_combine_body, relu=relu),
        grid=(N // R,),
        in_specs=[pl.BlockSpec((NC, R, AW), lambda i: (0, i, 0))],
        out_specs=pl.BlockSpec((R, D), lambda i: (i, 0)),
        out_shape=jax.ShapeDtypeStruct((N, D), jnp.float32),
    )(acc)


def _idx_fires(src_hbm, dst_hbm, ea_hbm, srcb3, dstb3, eab3, sem_idx, slot, base):
    pltpu.async_copy(src_hbm.at[pl.ds(base, B)], srcb3.at[slot], sem_idx)
    pltpu.async_copy(dst_hbm.at[pl.ds(base, B)], dstb3.at[slot], sem_idx)
    pltpu.async_copy(ea_hbm.at[pl.ds(base, B)], eab3.at[slot], sem_idx)


def _idx_waits(src_hbm, dst_hbm, ea_hbm, srcb3, dstb3, eab3, sem_idx, slot):
    pltpu.make_async_copy(src_hbm.at[pl.ds(0, B)], srcb3.at[slot], sem_idx).wait()
    pltpu.make_async_copy(dst_hbm.at[pl.ds(0, B)], dstb3.at[slot], sem_idx).wait()
    pltpu.make_async_copy(ea_hbm.at[pl.ds(0, B)], eab3.at[slot], sem_idx).wait()


def _sc_body(qp_hbm, kv_hbm, src_hbm, dst_hbm, ea_hbm, out_hbm,
             agg_sh, srcb3, dstb3, eab3, qr2, kr2, w, logb, sbuf,
             sem_idx, sem_row):
    cid = lax.axis_index("c")
    sid = lax.axis_index("s")
    wid = sid * NC + cid

    @pl.loop(0, B)
    def _zero_w(i):
        for j in range(AW // L):
            w[i, pl.ds(j * L, L)] = jnp.zeros((L,), jnp.float32)

    row0 = sid * (AGG_N // NS)
    for c in range(AGG_N // NS // B):
        pltpu.sync_copy(w, agg_sh.at[pl.ds(row0 + c * B, B)])

    plsc.subcore_barrier()

    nchunk = jnp.where(wid < NCHUNK - EXTRA0, FULL_T + 1, FULL_T)

    def chunk_id(t):
        return jnp.where(t < FULL_T, wid + t * NW, EXTRA0 + wid)

    lanes = lax.broadcasted_iota(jnp.int32, (L,), 0)
    last_lane = lanes == (L - 1)
    lane0 = lanes == 0

    def compute_chunk(qr, kr, eab):
        @pl.loop(0, B, unroll=4)
        def _dote(e):
            acc0 = qr[e, pl.ds(0, L)] * kr[e, pl.ds(0, L)]
            acc1 = qr[e, pl.ds(L, L)] * kr[e, pl.ds(L, L)]
            for c in range(2, D // L):
                t = qr[e, pl.ds(c * L, L)] * kr[e, pl.ds(c * L, L)]
                if c % 2 == 0:
                    acc0 = acc0 + t
                else:
                    acc1 = acc1 + t
            cum = plsc.cumsum(acc0 + acc1)
            ev = jnp.full((L,), e, jnp.int32)
            plsc.store_scatter(logb, [ev], cum, mask=last_lane)

        for g in range(B // L):
            rows = lanes + g * L
            qe0 = plsc.load_gather(qr, [rows, jnp.full((L,), D, jnp.int32)])
            qe1 = plsc.load_gather(qr, [rows, jnp.full((L,), D + 1, jnp.int32)])
            ea0 = plsc.load_gather(eab, [rows, jnp.zeros((L,), jnp.int32)])
            ea1 = plsc.load_gather(eab, [rows, jnp.full((L,), 1, jnp.int32)])
            lv = logb[pl.ds(g * L, L)] + qe0 * ea0 + qe1 * ea1
            sbuf[pl.ds(g * L, L)] = jnp.exp(lv)

        @pl.loop(0, B, unroll=4)
        def _wcols(e):
            sv = sbuf[pl.ds(e, L)]
            se = sv[0]
            plsc.store_scatter(
                w, [jnp.full((L,), e, jnp.int32), jnp.full((L,), D, jnp.int32)],
                sv, mask=lane0)
            for c in range(D // L):
                w[e, pl.ds(c * L, L)] = se * kr[e, pl.ds(D + c * L, L)]

    pltpu.sync_copy(src_hbm.at[pl.ds(wid * B, B)], srcb3.at[0])
    pltpu.sync_copy(dst_hbm.at[pl.ds(wid * B, B)], dstb3.at[0])
    pltpu.sync_copy(ea_hbm.at[pl.ds(wid * B, B)], eab3.at[0])
    pltpu.async_copy(qp_hbm.at[dstb3.at[0]], qr2[0], sem_row)
    pltpu.async_copy(kv_hbm.at[srcb3.at[0]], kr2[0], sem_row)
    _idx_fires(src_hbm, dst_hbm, ea_hbm, srcb3, dstb3, eab3, sem_idx, 1,
               chunk_id(1) * B)

    n_outer = (FULL_T + 1 + 5) // 6

    @pl.loop(0, n_outer)
    def _outer(t6):
        for i in range(6):
            t = t6 * 6 + i
            p2, p3 = i % 2, i % 3
            n2, n3 = (i + 1) % 2, (i + 1) % 3
            m3 = (i + 2) % 3

            @pl.when(t + 1 < nchunk)
            def _():
                _idx_waits(src_hbm, dst_hbm, ea_hbm, srcb3, dstb3, eab3,
                           sem_idx, n3)
                pltpu.async_copy(qp_hbm.at[dstb3.at[n3]], qr2[n2], sem_row)
                pltpu.async_copy(kv_hbm.at[srcb3.at[n3]], kr2[n2], sem_row)

            @pl.when(t + 2 < nchunk)
            def _():
                _idx_fires(src_hbm, dst_hbm, ea_hbm, srcb3, dstb3, eab3,
                           sem_idx, m3, chunk_id(t + 2) * B)

            @pl.when(t < nchunk)
            def _():
                pltpu.make_async_copy(qp_hbm.at[dstb3.at[p3]], qr2[p2],
                                      sem_row).wait()
                pltpu.make_async_copy(kv_hbm.at[srcb3.at[p3]], kr2[p2],
                                      sem_row).wait()
                compute_chunk(qr2[p2], kr2[p2], eab3.at[p3])
                pltpu.sync_copy(w, agg_sh.at[dstb3.at[p3]], add=True)

    plsc.subcore_barrier()
    pltpu.sync_copy(agg_sh.at[pl.ds(row0, AGG_N // NS)],
                    out_hbm.at[cid, pl.ds(row0, AGG_N // NS)])


@functools.partial(
    pl.kernel,
    out_type=jax.ShapeDtypeStruct((NC, AGG_N, AW), jnp.float32),
    mesh=plsc.VectorSubcoreMesh(core_axis_name="c", subcore_axis_name="s",
                                num_cores=NC, num_subcores=NS),
    compiler_params=pltpu.CompilerParams(use_tc_tiling_on_sc=False,
                                         needs_layout_passes=False,
                                         disable_bounds_checks=True),
    scratch_types=[
        pltpu.VMEM_SHARED((AGG_N, AW), jnp.float32),
        pltpu.VMEM((3, B), jnp.int32),
        pltpu.VMEM((3, B), jnp.int32),
        pltpu.VMEM((3, B, 2), jnp.float32),
        pltpu.VMEM((B, QW), jnp.float32),
        pltpu.VMEM((B, QW), jnp.float32),
        pltpu.VMEM((B, KVW), jnp.float32),
        pltpu.VMEM((B, KVW), jnp.float32),
        pltpu.VMEM((B, AW), jnp.float32),
        pltpu.VMEM((B,), jnp.float32),
        pltpu.VMEM((B + L,), jnp.float32),
        pltpu.SemaphoreType.DMA,
        pltpu.SemaphoreType.DMA,
    ],
)
def _sc_layer(qp_hbm, kv_hbm, src_hbm, dst_hbm, ea_hbm, out_hbm,
              agg_sh, srcb3, dstb3, eab3, qra, qrb, kra, krb, w, logb, sbuf,
              sem_idx, sem_row):
    _sc_body(qp_hbm, kv_hbm, src_hbm, dst_hbm, ea_hbm, out_hbm,
             agg_sh, srcb3, dstb3, eab3, [qra, qrb], [kra, krb], w, logb, sbuf,
             sem_idx, sem_row)


def kernel(x, edge_index, edge_attr, Wq, Wk, Wv, We):
    src = edge_index[0]
    dst = edge_index[1]
    n_layers = Wq.shape[0]
    h = x
    for l in range(n_layers):
        wet = jnp.zeros((D, QW - D), jnp.float32).at[:, :2].set(We[l].T)
        qp, kv = _prep(h, Wq[l], Wk[l], Wv[l], wet)
        acc = _sc_layer(qp, kv, src, dst, edge_attr)
        h = _combine(acc, relu=(l < n_layers - 1))
    return h


# speedup vs baseline: 12.1748x; 1.0115x over previous
import functools
import math

import jax
import jax.numpy as jnp
from jax import lax
from jax.experimental import pallas as pl
from jax.experimental.pallas import tpu as pltpu
from jax.experimental.pallas import tpu_sc as plsc

N = 10000
E = 320000
D = 128
QW = 144
KVW = 256
AW = 144
NC, NS, L = 2, 16, 16
AGG_N = 10240
NW = NC * NS
B = 32
NCHUNK = E // B
FULL_T = NCHUNK // NW
EXTRA0 = FULL_T * NW


def _prep_body(h_ref, wq_ref, wk_ref, wv_ref, wet_ref, qp_ref, kv_ref):
    h = h_ref[...]
    q = jnp.dot(h, wq_ref[...], preferred_element_type=jnp.float32)
    q = q * (1.0 / math.sqrt(D))
    qp_ref[:, :D] = q
    qp_ref[:, D:] = jnp.dot(q, wet_ref[...], preferred_element_type=jnp.float32)
    kv_ref[:, :D] = jnp.dot(h, wk_ref[...], preferred_element_type=jnp.float32)
    kv_ref[:, D:] = jnp.dot(h, wv_ref[...], preferred_element_type=jnp.float32)


def _prep(h, wq, wk, wv, wet):
    R = 1000
    return pl.pallas_call(
        _prep_body,
        grid=(N // R,),
        in_specs=[
            pl.BlockSpec((R, D), lambda i: (i, 0)),
            pl.BlockSpec((D, D), lambda i: (0, 0)),
            pl.BlockSpec((D, D), lambda i: (0, 0)),
            pl.BlockSpec((D, D), lambda i: (0, 0)),
            pl.BlockSpec((D, QW - D), lambda i: (0, 0)),
        ],
        out_specs=[
            pl.BlockSpec((R, QW), lambda i: (i, 0)),
            pl.BlockSpec((R, KVW), lambda i: (i, 0)),
        ],
        out_shape=[
            jax.ShapeDtypeStruct((N, QW), jnp.float32),
            jax.ShapeDtypeStruct((N, KVW), jnp.float32),
        ],
    )(h, wq, wk, wv, wet)


def _combine_body(acc_ref, out_ref, *, relu):
    a = acc_ref[0] + acc_ref[1]
    den = a[:, D:D + 1] + 1e-30
    h = a[:, :D] / den
    if relu:
        h = jnp.maximum(h, 0.0)
    out_ref[...] = h


def _combine(acc, relu):
    R = 1000
    return pl.pallas_call(
        functools.partial(_combine_body, relu=relu),
        grid=(N // R,),
        in_specs=[pl.BlockSpec((NC, R, AW), lambda i: (0, i, 0))],
        out_specs=pl.BlockSpec((R, D), lambda i: (i, 0)),
        out_shape=jax.ShapeDtypeStruct((N, D), jnp.float32),
    )(acc)


def _idx_fires(src_hbm, dst_hbm, ea_hbm, srcb3, dstb3, eab3, sem_idx, slot, base):
    pltpu.async_copy(src_hbm.at[pl.ds(base, B)], srcb3.at[slot], sem_idx)
    pltpu.async_copy(dst_hbm.at[pl.ds(base, B)], dstb3.at[slot], sem_idx)
    pltpu.async_copy(ea_hbm.at[pl.ds(base, B)], eab3.at[slot], sem_idx)


def _idx_waits(src_hbm, dst_hbm, ea_hbm, srcb3, dstb3, eab3, sem_idx, slot):
    pltpu.make_async_copy(src_hbm.at[pl.ds(0, B)], srcb3.at[slot], sem_idx).wait()
    pltpu.make_async_copy(dst_hbm.at[pl.ds(0, B)], dstb3.at[slot], sem_idx).wait()
    pltpu.make_async_copy(ea_hbm.at[pl.ds(0, B)], eab3.at[slot], sem_idx).wait()


def _sc_body(qp_hbm, kv_hbm, src_hbm, dst_hbm, ea_hbm, out_hbm,
             agg_sh, srcb3, dstb3, eab3, qr2, kr2, w, logb, sbuf,
             sem_idx, sem_row):
    cid = lax.axis_index("c")
    sid = lax.axis_index("s")
    wid = sid * NC + cid

    @pl.loop(0, B)
    def _zero_w(i):
        for j in range(AW // L):
            w[i, pl.ds(j * L, L)] = jnp.zeros((L,), jnp.float32)

    row0 = sid * (AGG_N // NS)
    for c in range(AGG_N // NS // B):
        pltpu.sync_copy(w, agg_sh.at[pl.ds(row0 + c * B, B)])

    plsc.subcore_barrier()

    nchunk = jnp.where(wid < NCHUNK - EXTRA0, FULL_T + 1, FULL_T)

    def chunk_id(t):
        return jnp.where(t < FULL_T, wid + t * NW, EXTRA0 + wid)

    lanes = lax.broadcasted_iota(jnp.int32, (L,), 0)
    last_lane = lanes == (L - 1)
    lane0 = lanes == 0

    def compute_chunk(qr, kr, eab):
        @pl.loop(0, B, unroll=4)
        def _dote(e):
            acc0 = qr[e, pl.ds(0, L)] * kr[e, pl.ds(0, L)]
            acc1 = qr[e, pl.ds(L, L)] * kr[e, pl.ds(L, L)]
            for c in range(2, D // L):
                t = qr[e, pl.ds(c * L, L)] * kr[e, pl.ds(c * L, L)]
                if c % 2 == 0:
                    acc0 = acc0 + t
                else:
                    acc1 = acc1 + t
            cum = plsc.cumsum(acc0 + acc1)
            ev = jnp.full((L,), e, jnp.int32)
            plsc.store_scatter(logb, [ev], cum, mask=last_lane)

        for g in range(B // L):
            rows = lanes + g * L
            qe0 = plsc.load_gather(qr, [rows, jnp.full((L,), D, jnp.int32)])
            qe1 = plsc.load_gather(qr, [rows, jnp.full((L,), D + 1, jnp.int32)])
            ea0 = plsc.load_gather(eab, [rows, jnp.zeros((L,), jnp.int32)])
            ea1 = plsc.load_gather(eab, [rows, jnp.full((L,), 1, jnp.int32)])
            lv = logb[pl.ds(g * L, L)] + qe0 * ea0 + qe1 * ea1
            sbuf[pl.ds(g * L, L)] = jnp.exp(lv)

        @pl.loop(0, B, unroll=4)
        def _wcols(e):
            sv = sbuf[pl.ds(e, L)]
            se = sv[0]
            plsc.store_scatter(
                w, [jnp.full((L,), e, jnp.int32), jnp.full((L,), D, jnp.int32)],
                sv, mask=lane0)
            vals = [kr[e, pl.ds(D + c * L, L)] for c in range(D // L)]
            prods = [se * v for v in vals]
            for c in range(D // L):
                w[e, pl.ds(c * L, L)] = prods[c]

    pltpu.sync_copy(src_hbm.at[pl.ds(wid * B, B)], srcb3.at[0])
    pltpu.sync_copy(dst_hbm.at[pl.ds(wid * B, B)], dstb3.at[0])
    pltpu.sync_copy(ea_hbm.at[pl.ds(wid * B, B)], eab3.at[0])
    pltpu.async_copy(qp_hbm.at[dstb3.at[0]], qr2[0], sem_row)
    pltpu.async_copy(kv_hbm.at[srcb3.at[0]], kr2[0], sem_row)
    _idx_fires(src_hbm, dst_hbm, ea_hbm, srcb3, dstb3, eab3, sem_idx, 1,
               chunk_id(1) * B)

    n_outer = (FULL_T + 1 + 5) // 6

    @pl.loop(0, n_outer)
    def _outer(t6):
        for i in range(6):
            t = t6 * 6 + i
            p2, p3 = i % 2, i % 3
            n2, n3 = (i + 1) % 2, (i + 1) % 3
            m3 = (i + 2) % 3

            @pl.when(t + 1 < nchunk)
            def _():
                _idx_waits(src_hbm, dst_hbm, ea_hbm, srcb3, dstb3, eab3,
                           sem_idx, n3)
                pltpu.async_copy(qp_hbm.at[dstb3.at[n3]], qr2[n2], sem_row)
                pltpu.async_copy(kv_hbm.at[srcb3.at[n3]], kr2[n2], sem_row)

            @pl.when(t + 2 < nchunk)
            def _():
                _idx_fires(src_hbm, dst_hbm, ea_hbm, srcb3, dstb3, eab3,
                           sem_idx, m3, chunk_id(t + 2) * B)

            @pl.when(t < nchunk)
            def _():
                pltpu.make_async_copy(qp_hbm.at[dstb3.at[p3]], qr2[p2],
                                      sem_row).wait()
                pltpu.make_async_copy(kv_hbm.at[srcb3.at[p3]], kr2[p2],
                                      sem_row).wait()
                compute_chunk(qr2[p2], kr2[p2], eab3.at[p3])
                pltpu.sync_copy(w, agg_sh.at[dstb3.at[p3]], add=True)

    plsc.subcore_barrier()
    pltpu.sync_copy(agg_sh.at[pl.ds(row0, AGG_N // NS)],
                    out_hbm.at[cid, pl.ds(row0, AGG_N // NS)])


@functools.partial(
    pl.kernel,
    out_type=jax.ShapeDtypeStruct((NC, AGG_N, AW), jnp.float32),
    mesh=plsc.VectorSubcoreMesh(core_axis_name="c", subcore_axis_name="s",
                                num_cores=NC, num_subcores=NS),
    compiler_params=pltpu.CompilerParams(use_tc_tiling_on_sc=False,
                                         needs_layout_passes=False,
                                         disable_bounds_checks=True),
    scratch_types=[
        pltpu.VMEM_SHARED((AGG_N, AW), jnp.float32),
        pltpu.VMEM((3, B), jnp.int32),
        pltpu.VMEM((3, B), jnp.int32),
        pltpu.VMEM((3, B, 2), jnp.float32),
        pltpu.VMEM((B, QW), jnp.float32),
        pltpu.VMEM((B, QW), jnp.float32),
        pltpu.VMEM((B, KVW), jnp.float32),
        pltpu.VMEM((B, KVW), jnp.float32),
        pltpu.VMEM((B, AW), jnp.float32),
        pltpu.VMEM((B,), jnp.float32),
        pltpu.VMEM((B + L,), jnp.float32),
        pltpu.SemaphoreType.DMA,
        pltpu.SemaphoreType.DMA,
    ],
)
def _sc_layer(qp_hbm, kv_hbm, src_hbm, dst_hbm, ea_hbm, out_hbm,
              agg_sh, srcb3, dstb3, eab3, qra, qrb, kra, krb, w, logb, sbuf,
              sem_idx, sem_row):
    _sc_body(qp_hbm, kv_hbm, src_hbm, dst_hbm, ea_hbm, out_hbm,
             agg_sh, srcb3, dstb3, eab3, [qra, qrb], [kra, krb], w, logb, sbuf,
             sem_idx, sem_row)


def kernel(x, edge_index, edge_attr, Wq, Wk, Wv, We):
    src = edge_index[0]
    dst = edge_index[1]
    n_layers = Wq.shape[0]
    h = x
    for l in range(n_layers):
        wet = jnp.zeros((D, QW - D), jnp.float32).at[:, :2].set(We[l].T)
        qp, kv = _prep(h, Wq[l], Wk[l], Wv[l], wet)
        acc = _sc_layer(qp, kv, src, dst, edge_attr)
        h = _combine(acc, relu=(l < n_layers - 1))
    return h


# speedup vs baseline: 13.1225x; 1.0778x over previous
import functools
import math

import jax
import jax.numpy as jnp
from jax import lax
from jax.experimental import pallas as pl
from jax.experimental.pallas import tpu as pltpu
from jax.experimental.pallas import tpu_sc as plsc

N = 10000
E = 320000
D = 128
QW = 144
KVW = 256
AW = 144
NC, NS, L = 2, 16, 16
AGG_N = 10240
NW = NC * NS
B = 32
NCHUNK = E // B
FULL_T = NCHUNK // NW
EXTRA0 = FULL_T * NW


def _prep_body(h_ref, wq_ref, wk_ref, wv_ref, wet_ref, qp_ref, kv_ref):
    h = h_ref[...]
    q = jnp.dot(h, wq_ref[...], preferred_element_type=jnp.float32)
    q = q * (1.0 / math.sqrt(D))
    qp_ref[:, :D] = q
    qp_ref[:, D:] = jnp.dot(q, wet_ref[...], preferred_element_type=jnp.float32)
    kv_ref[:, :D] = jnp.dot(h, wk_ref[...], preferred_element_type=jnp.float32)
    kv_ref[:, D:] = jnp.dot(h, wv_ref[...], preferred_element_type=jnp.float32)


def _prep(h, wq, wk, wv, wet):
    R = 1000
    return pl.pallas_call(
        _prep_body,
        grid=(N // R,),
        in_specs=[
            pl.BlockSpec((R, D), lambda i: (i, 0)),
            pl.BlockSpec((D, D), lambda i: (0, 0)),
            pl.BlockSpec((D, D), lambda i: (0, 0)),
            pl.BlockSpec((D, D), lambda i: (0, 0)),
            pl.BlockSpec((D, QW - D), lambda i: (0, 0)),
        ],
        out_specs=[
            pl.BlockSpec((R, QW), lambda i: (i, 0)),
            pl.BlockSpec((R, KVW), lambda i: (i, 0)),
        ],
        out_shape=[
            jax.ShapeDtypeStruct((N, QW), jnp.float32),
            jax.ShapeDtypeStruct((N, KVW), jnp.float32),
        ],
    )(h, wq, wk, wv, wet)


def _fused_body(acc_ref, wq_ref, wk_ref, wv_ref, wet_ref, qp_ref, kv_ref):
    a = acc_ref[0] + acc_ref[1]
    den = a[:, D:D + 1] + 1e-30
    h = jnp.maximum(a[:, :D] / den, 0.0)
    q = jnp.dot(h, wq_ref[...], preferred_element_type=jnp.float32)
    q = q * (1.0 / math.sqrt(D))
    qp_ref[:, :D] = q
    qp_ref[:, D:] = jnp.dot(q, wet_ref[...], preferred_element_type=jnp.float32)
    kv_ref[:, :D] = jnp.dot(h, wk_ref[...], preferred_element_type=jnp.float32)
    kv_ref[:, D:] = jnp.dot(h, wv_ref[...], preferred_element_type=jnp.float32)


def _fused(acc, wq, wk, wv, wet):
    R = 1000
    return pl.pallas_call(
        _fused_body,
        grid=(N // R,),
        in_specs=[
            pl.BlockSpec((NC, R, AW), lambda i: (0, i, 0)),
            pl.BlockSpec((D, D), lambda i: (0, 0)),
            pl.BlockSpec((D, D), lambda i: (0, 0)),
            pl.BlockSpec((D, D), lambda i: (0, 0)),
            pl.BlockSpec((D, QW - D), lambda i: (0, 0)),
        ],
        out_specs=[
            pl.BlockSpec((R, QW), lambda i: (i, 0)),
            pl.BlockSpec((R, KVW), lambda i: (i, 0)),
        ],
        out_shape=[
            jax.ShapeDtypeStruct((N, QW), jnp.float32),
            jax.ShapeDtypeStruct((N, KVW), jnp.float32),
        ],
    )(acc, wq, wk, wv, wet)


def _combine_body(acc_ref, out_ref, *, relu):
    a = acc_ref[0] + acc_ref[1]
    den = a[:, D:D + 1] + 1e-30
    h = a[:, :D] / den
    if relu:
        h = jnp.maximum(h, 0.0)
    out_ref[...] = h


def _combine(acc, relu):
    R = 1000
    return pl.pallas_call(
        functools.partial(_combine_body, relu=relu),
        grid=(N // R,),
        in_specs=[pl.BlockSpec((NC, R, AW), lambda i: (0, i, 0))],
        out_specs=pl.BlockSpec((R, D), lambda i: (i, 0)),
        out_shape=jax.ShapeDtypeStruct((N, D), jnp.float32),
    )(acc)


def _idx_fires(src_hbm, dst_hbm, ea_hbm, srcb3, dstb3, eab3, sem_idx, slot, base):
    pltpu.async_copy(src_hbm.at[pl.ds(base, B)], srcb3.at[slot], sem_idx)
    pltpu.async_copy(dst_hbm.at[pl.ds(base, B)], dstb3.at[slot], sem_idx)
    pltpu.async_copy(ea_hbm.at[pl.ds(base, B)], eab3.at[slot], sem_idx)


def _idx_waits(src_hbm, dst_hbm, ea_hbm, srcb3, dstb3, eab3, sem_idx, slot):
    pltpu.make_async_copy(src_hbm.at[pl.ds(0, B)], srcb3.at[slot], sem_idx).wait()
    pltpu.make_async_copy(dst_hbm.at[pl.ds(0, B)], dstb3.at[slot], sem_idx).wait()
    pltpu.make_async_copy(ea_hbm.at[pl.ds(0, B)], eab3.at[slot], sem_idx).wait()


def _sc_body(qp_hbm, kv_hbm, src_hbm, dst_hbm, ea_hbm, out_hbm,
             agg_sh, srcb3, dstb3, eab3, qr2, kr2, w2, dstw2, logb, sbuf,
             sem_idx, sem_row, sem_sc):
    cid = lax.axis_index("c")
    sid = lax.axis_index("s")
    wid = sid * NC + cid

    for w in w2:
        @pl.loop(0, B)
        def _zero_w(i):
            for j in range(AW // L):
                w[i, pl.ds(j * L, L)] = jnp.zeros((L,), jnp.float32)

    row0 = sid * (AGG_N // NS)
    for c in range(AGG_N // NS // B):
        pltpu.sync_copy(w2[c % 2], agg_sh.at[pl.ds(row0 + c * B, B)])

    plsc.subcore_barrier()

    nchunk = jnp.where(wid < NCHUNK - EXTRA0, FULL_T + 1, FULL_T)

    def chunk_id(t):
        return jnp.where(t < FULL_T, wid + t * NW, EXTRA0 + wid)

    lanes = lax.broadcasted_iota(jnp.int32, (L,), 0)
    last_lane = lanes == (L - 1)
    lane0 = lanes == 0

    def compute_chunk(qr, kr, eab, w):
        @pl.loop(0, B, unroll=4)
        def _dote(e):
            acc0 = qr[e, pl.ds(0, L)] * kr[e, pl.ds(0, L)]
            acc1 = qr[e, pl.ds(L, L)] * kr[e, pl.ds(L, L)]
            for c in range(2, D // L):
                t = qr[e, pl.ds(c * L, L)] * kr[e, pl.ds(c * L, L)]
                if c % 2 == 0:
                    acc0 = acc0 + t
                else:
                    acc1 = acc1 + t
            cum = plsc.cumsum(acc0 + acc1)
            ev = jnp.full((L,), e, jnp.int32)
            plsc.store_scatter(logb, [ev], cum, mask=last_lane)

        for g in range(B // L):
            rows = lanes + g * L
            qe0 = plsc.load_gather(qr, [rows, jnp.full((L,), D, jnp.int32)])
            qe1 = plsc.load_gather(qr, [rows, jnp.full((L,), D + 1, jnp.int32)])
            ea0 = plsc.load_gather(eab, [rows, jnp.zeros((L,), jnp.int32)])
            ea1 = plsc.load_gather(eab, [rows, jnp.full((L,), 1, jnp.int32)])
            lv = logb[pl.ds(g * L, L)] + qe0 * ea0 + qe1 * ea1
            sbuf[pl.ds(g * L, L)] = jnp.exp(lv)

        @pl.loop(0, B, unroll=4)
        def _wcols(e):
            sv = sbuf[pl.ds(e, L)]
            se = sv[0]
            plsc.store_scatter(
                w, [jnp.full((L,), e, jnp.int32), jnp.full((L,), D, jnp.int32)],
                sv, mask=lane0)
            vals = [kr[e, pl.ds(D + c * L, L)] for c in range(D // L)]
            prods = [se * v for v in vals]
            for c in range(D // L):
                w[e, pl.ds(c * L, L)] = prods[c]

    pltpu.sync_copy(src_hbm.at[pl.ds(wid * B, B)], srcb3.at[0])
    pltpu.sync_copy(dst_hbm.at[pl.ds(wid * B, B)], dstb3.at[0])
    pltpu.sync_copy(ea_hbm.at[pl.ds(wid * B, B)], eab3.at[0])
    pltpu.async_copy(qp_hbm.at[dstb3.at[0]], qr2[0], sem_row)
    pltpu.async_copy(kv_hbm.at[srcb3.at[0]], kr2[0], sem_row)
    _idx_fires(src_hbm, dst_hbm, ea_hbm, srcb3, dstb3, eab3, sem_idx, 1,
               chunk_id(1) * B)

    n_outer = (FULL_T + 1 + 5) // 6

    @pl.loop(0, n_outer)
    def _outer(t6):
        for i in range(6):
            t = t6 * 6 + i
            p2, p3 = i % 2, i % 3
            n2, n3 = (i + 1) % 2, (i + 1) % 3
            m3 = (i + 2) % 3

            @pl.when(t + 1 < nchunk)
            def _():
                _idx_waits(src_hbm, dst_hbm, ea_hbm, srcb3, dstb3, eab3,
                           sem_idx, n3)
                pltpu.async_copy(qp_hbm.at[dstb3.at[n3]], qr2[n2], sem_row)
                pltpu.async_copy(kv_hbm.at[srcb3.at[n3]], kr2[n2], sem_row)

            @pl.when(t + 2 < nchunk)
            def _():
                _idx_fires(src_hbm, dst_hbm, ea_hbm, srcb3, dstb3, eab3,
                           sem_idx, m3, chunk_id(t + 2) * B)

            @pl.when(t < nchunk)
            def _():
                pltpu.make_async_copy(qp_hbm.at[dstb3.at[p3]], qr2[p2],
                                      sem_row).wait()
                pltpu.make_async_copy(kv_hbm.at[srcb3.at[p3]], kr2[p2],
                                      sem_row).wait()

                @pl.when(t >= 2)
                def _():
                    pltpu.make_async_copy(w2[p2], agg_sh.at[dstw2.at[p2]],
                                          sem_sc).wait()

                compute_chunk(qr2[p2], kr2[p2], eab3.at[p3], w2[p2])

                for j in range(B // L):
                    dstw2[p2, pl.ds(j * L, L)] = dstb3[p3, pl.ds(j * L, L)]
                pltpu.async_copy(w2[p2], agg_sh.at[dstw2.at[p2]], sem_sc,
                                 add=True)

    pltpu.make_async_copy(w2[0], agg_sh.at[dstw2.at[0]], sem_sc).wait()
    pltpu.make_async_copy(w2[1], agg_sh.at[dstw2.at[1]], sem_sc).wait()

    plsc.subcore_barrier()
    pltpu.sync_copy(agg_sh.at[pl.ds(row0, AGG_N // NS)],
                    out_hbm.at[cid, pl.ds(row0, AGG_N // NS)])


@functools.partial(
    pl.kernel,
    out_type=jax.ShapeDtypeStruct((NC, AGG_N, AW), jnp.float32),
    mesh=plsc.VectorSubcoreMesh(core_axis_name="c", subcore_axis_name="s",
                                num_cores=NC, num_subcores=NS),
    compiler_params=pltpu.CompilerParams(use_tc_tiling_on_sc=False,
                                         needs_layout_passes=False,
                                         disable_bounds_checks=True),
    scratch_types=[
        pltpu.VMEM_SHARED((AGG_N, AW), jnp.float32),
        pltpu.VMEM((3, B), jnp.int32),
        pltpu.VMEM((3, B), jnp.int32),
        pltpu.VMEM((3, B, 2), jnp.float32),
        pltpu.VMEM((B, QW), jnp.float32),
        pltpu.VMEM((B, QW), jnp.float32),
        pltpu.VMEM((B, KVW), jnp.float32),
        pltpu.VMEM((B, KVW), jnp.float32),
        pltpu.VMEM((B, AW), jnp.float32),
        pltpu.VMEM((B, AW), jnp.float32),
        pltpu.VMEM((2, B), jnp.int32),
        pltpu.VMEM((B,), jnp.float32),
        pltpu.VMEM((B + L,), jnp.float32),
        pltpu.SemaphoreType.DMA,
        pltpu.SemaphoreType.DMA,
        pltpu.SemaphoreType.DMA,
    ],
)
def _sc_layer(qp_hbm, kv_hbm, src_hbm, dst_hbm, ea_hbm, out_hbm,
              agg_sh, srcb3, dstb3, eab3, qra, qrb, kra, krb, wa, wb, dstw2,
              logb, sbuf, sem_idx, sem_row, sem_sc):
    _sc_body(qp_hbm, kv_hbm, src_hbm, dst_hbm, ea_hbm, out_hbm,
             agg_sh, srcb3, dstb3, eab3, [qra, qrb], [kra, krb], [wa, wb],
             dstw2, logb, sbuf, sem_idx, sem_row, sem_sc)


def kernel(x, edge_index, edge_attr, Wq, Wk, Wv, We):
    src = edge_index[0]
    dst = edge_index[1]
    n_layers = Wq.shape[0]
    wets = [jnp.zeros((D, QW - D), jnp.float32).at[:, :2].set(We[l].T)
            for l in range(n_layers)]
    qp, kv = _prep(x, Wq[0], Wk[0], Wv[0], wets[0])
    acc = None
    for l in range(n_layers):
        acc = _sc_layer(qp, kv, src, dst, edge_attr)
        if l < n_layers - 1:
            qp, kv = _fused(acc, Wq[l + 1], Wk[l + 1], Wv[l + 1], wets[l + 1])
    return _combine(acc, relu=False)
